# baseline (speedup 1.0000x reference)
import functools
import math

import jax
import jax.numpy as jnp
from jax import lax
from jax.experimental import pallas as pl
from jax.experimental.pallas import tpu as pltpu
from jax.experimental.pallas import tpu_sc as plsc

VOCAB = 128
EMB = 128
B = 4096
S = 200
PAD = 0

NC = 2
NS = 16
BL0 = 104
BL1 = S - BL0
P = 80
FR = P * VOCAB
SEQ_PER_TILE = B // NS
NBUF = 2
CH = 32
NCH = SEQ_PER_TILE // CH


def _build_body(tokens_ref, table_ref, f_ref, mask_ref, idx0_ref, idx1_ref):
    tab = table_ref[...]
    row_ids = lax.broadcasted_iota(jnp.int32, (VOCAB, EMB), 0)
    tab = jnp.where(row_ids == PAD, 0.0, tab)

    pos = lax.broadcasted_iota(jnp.int32, (S, EMB), 0).astype(jnp.float32)
    d = lax.broadcasted_iota(jnp.int32, (S, EMB), 1)
    dt = jnp.exp(((d // 2) * 2).astype(jnp.float32) * (-math.log(10000.0) / EMB))
    ang = pos * dt
    pe = jnp.where(d % 2 == 0, jnp.sin(ang), jnp.cos(ang))

    f_ref[...] = tab[None, :, :] + pe[:, None, :]

    tok = tokens_ref[...]
    mask_ref[...] = tok == PAD
    s_ids = lax.broadcasted_iota(jnp.int32, (B, S), 1)
    idx0_ref[...] = s_ids[:, :BL0] * VOCAB + tok[:, :BL0]
    loc1 = (s_ids[:, BL0:] - BL0) * VOCAB + tok[:, BL0:]
    col1 = lax.broadcasted_iota(jnp.int32, (B, BL1), 1)
    idx1 = jnp.where(col1 >= P, loc1 + BL0 * VOCAB, loc1)
    idx1_ref[...] = jnp.concatenate(
        [idx1, jnp.zeros((B, BL0 - BL1), jnp.int32)], axis=1
    )


def _build(tokens_idx, emb_table):
    return pl.pallas_call(
        _build_body,
        out_shape=[
            jax.ShapeDtypeStruct((S, VOCAB, EMB), jnp.float32),
            jax.ShapeDtypeStruct((B, S), jnp.bool_),
            jax.ShapeDtypeStruct((B, BL0), jnp.int32),
            jax.ShapeDtypeStruct((B, BL0), jnp.int32),
        ],
    )(tokens_idx, emb_table)


_sc_mesh = plsc.VectorSubcoreMesh(core_axis_name="c", subcore_axis_name="s")


@functools.partial(
    pl.kernel,
    out_type=jax.ShapeDtypeStruct((B * S, EMB), jnp.float32),
    mesh=_sc_mesh,
    scratch_types=[
        pltpu.VMEM_SHARED((FR, EMB), jnp.float32),
        pltpu.VMEM((2, CH, BL0), jnp.int32),
        pltpu.VMEM((NBUF, BL0, EMB), jnp.float32),
    ]
    + [pltpu.SemaphoreType.DMA] * (3 * NBUF + 2),
)
def _sc_gather(f_hbm, idx0_hbm, idx1_hbm, out_hbm, f_sh, idx_v, bufs, *sems):
    gsems = sems[:NBUF]
    hsems = sems[NBUF : 2 * NBUF]
    ssems = sems[2 * NBUF : 3 * NBUF]
    isems = sems[3 * NBUF :]
    cid = lax.axis_index("c")
    sid = lax.axis_index("s")

    def half(idx_hbm, blen, f_base, pos_lo):
        fill = FR // NS
        pltpu.sync_copy(
            f_hbm.at[pl.ds(f_base + sid * fill, fill)],
            f_sh.at[pl.ds(sid * fill, fill)],
        )

        def idx_load(c):
            pltpu.async_copy(
                idx_hbm.at[pl.ds(sid * SEQ_PER_TILE + c * CH, CH)],
                idx_v.at[c % 2],
                isems[c % 2],
            )

        def idx_wait(c):
            pltpu.make_async_copy(
                idx_hbm.at[pl.ds(0, CH)],
                idx_v.at[c % 2],
                isems[c % 2],
            ).wait()

        idx_load(0)
        plsc.subcore_barrier()

        def out_rows(g):
            return pl.ds((sid * SEQ_PER_TILE + g) * S + pos_lo, blen)

        def fetch_start(slot, c, j):
            pltpu.async_copy(
                f_sh.at[idx_v.at[c % 2, j, pl.ds(0, P)]],
                bufs.at[slot, pl.ds(0, P)],
                gsems[slot],
            )
            pltpu.async_copy(
                f_hbm.at[idx_v.at[c % 2, j, pl.ds(P, blen - P)]],
                bufs.at[slot, pl.ds(P, blen - P)],
                hsems[slot],
            )

        def fetch_wait(slot):
            pltpu.make_async_copy(
                f_sh.at[idx_v.at[0, 0, pl.ds(0, P)]],
                bufs.at[slot, pl.ds(0, P)],
                gsems[slot],
            ).wait()
            pltpu.make_async_copy(
                f_hbm.at[idx_v.at[0, 0, pl.ds(P, blen - P)]],
                bufs.at[slot, pl.ds(P, blen - P)],
                hsems[slot],
            ).wait()

        def scatter_start(slot, c, j):
            pltpu.async_copy(
                bufs.at[slot, pl.ds(0, blen)],
                out_hbm.at[out_rows(c * CH + j)],
                ssems[slot],
            )

        def scatter_wait(slot):
            pltpu.make_async_copy(
                bufs.at[slot, pl.ds(0, blen)], out_hbm.at[out_rows(0)], ssems[slot]
            ).wait()

        idx_wait(0)
        idx_load(1)
        for b in range(NBUF):
            fetch_start(b, 0, b)

        for c in range(NCH):
            @pl.loop(0, (CH - NBUF) // NBUF)
            def _(o):
                for b in range(NBUF):
                    j = o * NBUF + b
                    fetch_wait(b)
                    scatter_start(b, c, j)
                    scatter_wait(b)
                    fetch_start(b, c, j + NBUF)

            if c + 1 < NCH:
                idx_wait(c + 1)
            for b in range(NBUF):
                fetch_wait(b)
                scatter_start(b, c, CH - NBUF + b)
                scatter_wait(b)
                if c + 1 < NCH:
                    fetch_start(b, c + 1, b)
            if c + 2 < NCH:
                idx_load(c + 2)

    @pl.when(cid == 0)
    def _():
        half(idx0_hbm, BL0, 0, 0)

    @pl.when(cid == 1)
    def _():
        half(idx1_hbm, BL1, BL0 * VOCAB, BL0)


def kernel(tokens_idx, emb_table):
    f, mask, idx0, idx1 = _build(tokens_idx, emb_table)
    f_flat = f.reshape(S * VOCAB, EMB)
    out = _sc_gather(f_flat, idx0, idx1)
    return out.reshape(B, S, EMB), mask


# speedup vs baseline: 1.0608x; 1.0608x over previous
import functools
import math

import jax
import jax.numpy as jnp
from jax import lax
from jax.experimental import pallas as pl
from jax.experimental.pallas import tpu as pltpu
from jax.experimental.pallas import tpu_sc as plsc

VOCAB = 128
EMB = 128
B = 4096
S = 200
PAD = 0

NC = 2
NS = 16
BL0 = 104
BL1 = S - BL0
P = 93
FR = P * VOCAB
SEQ_PER_TILE = B // NS
NBUF = 2
CH = 32
NCH = SEQ_PER_TILE // CH


def _build_body(tokens_ref, table_ref, f_ref, mask_ref, idx0_ref, idx1_ref):
    tab = table_ref[...]
    row_ids = lax.broadcasted_iota(jnp.int32, (VOCAB, EMB), 0)
    tab = jnp.where(row_ids == PAD, 0.0, tab)

    pos = lax.broadcasted_iota(jnp.int32, (S, EMB), 0).astype(jnp.float32)
    d = lax.broadcasted_iota(jnp.int32, (S, EMB), 1)
    dt = jnp.exp(((d // 2) * 2).astype(jnp.float32) * (-math.log(10000.0) / EMB))
    ang = pos * dt
    pe = jnp.where(d % 2 == 0, jnp.sin(ang), jnp.cos(ang))

    f_ref[...] = tab[None, :, :] + pe[:, None, :]

    tok = tokens_ref[...]
    mask_ref[...] = tok == PAD
    s_ids = lax.broadcasted_iota(jnp.int32, (B, S), 1)
    idx0_ref[...] = s_ids[:, :BL0] * VOCAB + tok[:, :BL0]
    loc1 = (s_ids[:, BL0:] - BL0) * VOCAB + tok[:, BL0:]
    col1 = lax.broadcasted_iota(jnp.int32, (B, BL1), 1)
    idx1 = jnp.where(col1 >= P, loc1 + BL0 * VOCAB, loc1)
    idx1_ref[...] = jnp.concatenate(
        [idx1, jnp.zeros((B, BL0 - BL1), jnp.int32)], axis=1
    )


def _build(tokens_idx, emb_table):
    return pl.pallas_call(
        _build_body,
        out_shape=[
            jax.ShapeDtypeStruct((S, VOCAB, EMB), jnp.float32),
            jax.ShapeDtypeStruct((B, S), jnp.bool_),
            jax.ShapeDtypeStruct((B, BL0), jnp.int32),
            jax.ShapeDtypeStruct((B, BL0), jnp.int32),
        ],
    )(tokens_idx, emb_table)


_sc_mesh = plsc.VectorSubcoreMesh(core_axis_name="c", subcore_axis_name="s")


@functools.partial(
    pl.kernel,
    out_type=jax.ShapeDtypeStruct((B * S, EMB), jnp.float32),
    mesh=_sc_mesh,
    scratch_types=[
        pltpu.VMEM_SHARED((FR, EMB), jnp.float32),
        pltpu.VMEM((2, CH, BL0), jnp.int32),
        pltpu.VMEM((NBUF, BL0, EMB), jnp.float32),
    ]
    + [pltpu.SemaphoreType.DMA] * (3 * NBUF + 2),
)
def _sc_gather(f_hbm, idx0_hbm, idx1_hbm, out_hbm, f_sh, idx_v, bufs, *sems):
    gsems = sems[:NBUF]
    hsems = sems[NBUF : 2 * NBUF]
    ssems = sems[2 * NBUF : 3 * NBUF]
    isems = sems[3 * NBUF :]
    cid = lax.axis_index("c")
    sid = lax.axis_index("s")

    def half(idx_hbm, blen, f_base, pos_lo):
        fill = FR // NS
        pltpu.sync_copy(
            f_hbm.at[pl.ds(f_base + sid * fill, fill)],
            f_sh.at[pl.ds(sid * fill, fill)],
        )

        def idx_load(c):
            pltpu.async_copy(
                idx_hbm.at[pl.ds(sid * SEQ_PER_TILE + c * CH, CH)],
                idx_v.at[c % 2],
                isems[c % 2],
            )

        def idx_wait(c):
            pltpu.make_async_copy(
                idx_hbm.at[pl.ds(0, CH)],
                idx_v.at[c % 2],
                isems[c % 2],
            ).wait()

        idx_load(0)
        plsc.subcore_barrier()

        def out_rows(g):
            return pl.ds((sid * SEQ_PER_TILE + g) * S + pos_lo, blen)

        def fetch_start(slot, c, j):
            pltpu.async_copy(
                f_sh.at[idx_v.at[c % 2, j, pl.ds(0, P)]],
                bufs.at[slot, pl.ds(0, P)],
                gsems[slot],
            )
            pltpu.async_copy(
                f_hbm.at[idx_v.at[c % 2, j, pl.ds(P, blen - P)]],
                bufs.at[slot, pl.ds(P, blen - P)],
                hsems[slot],
            )

        def fetch_wait(slot):
            pltpu.make_async_copy(
                f_sh.at[idx_v.at[0, 0, pl.ds(0, P)]],
                bufs.at[slot, pl.ds(0, P)],
                gsems[slot],
            ).wait()
            pltpu.make_async_copy(
                f_hbm.at[idx_v.at[0, 0, pl.ds(P, blen - P)]],
                bufs.at[slot, pl.ds(P, blen - P)],
                hsems[slot],
            ).wait()

        def scatter_start(slot, c, j):
            pltpu.async_copy(
                bufs.at[slot, pl.ds(0, blen)],
                out_hbm.at[out_rows(c * CH + j)],
                ssems[slot],
            )

        def scatter_wait(slot):
            pltpu.make_async_copy(
                bufs.at[slot, pl.ds(0, blen)], out_hbm.at[out_rows(0)], ssems[slot]
            ).wait()

        idx_wait(0)
        idx_load(1)
        for b in range(NBUF):
            fetch_start(b, 0, b)

        for c in range(NCH):
            @pl.loop(0, (CH - NBUF) // NBUF)
            def _(o):
                for b in range(NBUF):
                    j = o * NBUF + b
                    fetch_wait(b)
                    scatter_start(b, c, j)
                    scatter_wait(b)
                    fetch_start(b, c, j + NBUF)

            if c + 1 < NCH:
                idx_wait(c + 1)
            for b in range(NBUF):
                fetch_wait(b)
                scatter_start(b, c, CH - NBUF + b)
                scatter_wait(b)
                if c + 1 < NCH:
                    fetch_start(b, c + 1, b)
            if c + 2 < NCH:
                idx_load(c + 2)

    @pl.when(cid == 0)
    def _():
        half(idx0_hbm, BL0, 0, 0)

    @pl.when(cid == 1)
    def _():
        half(idx1_hbm, BL1, BL0 * VOCAB, BL0)


def kernel(tokens_idx, emb_table):
    f, mask, idx0, idx1 = _build(tokens_idx, emb_table)
    f_flat = f.reshape(S * VOCAB, EMB)
    out = _sc_gather(f_flat, idx0, idx1)
    return out.reshape(B, S, EMB), mask


# speedup vs baseline: 1.1453x; 1.0797x over previous
import functools
import math

import jax
import jax.numpy as jnp
from jax import lax
from jax.experimental import pallas as pl
from jax.experimental.pallas import tpu as pltpu
from jax.experimental.pallas import tpu_sc as plsc

VOCAB = 128
EMB = 128
B = 4096
S = 200
PAD = 0

NC = 2
NS = 16
BL0 = 104
BL1 = S - BL0
P = 84
FR = P * VOCAB
SEQ_PER_TILE = B // NS
NBUF = 3
CH = 16
NCH = SEQ_PER_TILE // CH


def _build_body(tokens_ref, table_ref, f_ref, mask_ref, idx0_ref, idx1_ref):
    tab = table_ref[...]
    row_ids = lax.broadcasted_iota(jnp.int32, (VOCAB, EMB), 0)
    tab = jnp.where(row_ids == PAD, 0.0, tab)

    pos = lax.broadcasted_iota(jnp.int32, (S, EMB), 0).astype(jnp.float32)
    d = lax.broadcasted_iota(jnp.int32, (S, EMB), 1)
    dt = jnp.exp(((d // 2) * 2).astype(jnp.float32) * (-math.log(10000.0) / EMB))
    ang = pos * dt
    pe = jnp.where(d % 2 == 0, jnp.sin(ang), jnp.cos(ang))

    f_ref[...] = tab[None, :, :] + pe[:, None, :]

    tok = tokens_ref[...]
    mask_ref[...] = tok == PAD
    s_ids = lax.broadcasted_iota(jnp.int32, (B, S), 1)
    idx0_ref[...] = s_ids[:, :BL0] * VOCAB + tok[:, :BL0]
    loc1 = (s_ids[:, BL0:] - BL0) * VOCAB + tok[:, BL0:]
    col1 = lax.broadcasted_iota(jnp.int32, (B, BL1), 1)
    idx1 = jnp.where(col1 >= P, loc1 + BL0 * VOCAB, loc1)
    idx1_ref[...] = jnp.concatenate(
        [idx1, jnp.zeros((B, BL0 - BL1), jnp.int32)], axis=1
    )


def _build(tokens_idx, emb_table):
    return pl.pallas_call(
        _build_body,
        out_shape=[
            jax.ShapeDtypeStruct((S, VOCAB, EMB), jnp.float32),
            jax.ShapeDtypeStruct((B, S), jnp.bool_),
            jax.ShapeDtypeStruct((B, BL0), jnp.int32),
            jax.ShapeDtypeStruct((B, BL0), jnp.int32),
        ],
    )(tokens_idx, emb_table)


_sc_mesh = plsc.VectorSubcoreMesh(core_axis_name="c", subcore_axis_name="s")


@functools.partial(
    pl.kernel,
    out_type=jax.ShapeDtypeStruct((B * S, EMB), jnp.float32),
    mesh=_sc_mesh,
    scratch_types=[
        pltpu.VMEM_SHARED((FR, EMB), jnp.float32),
        pltpu.VMEM((2, CH, BL0), jnp.int32),
        pltpu.VMEM((NBUF, BL0, EMB), jnp.float32),
    ]
    + [pltpu.SemaphoreType.DMA] * (3 * NBUF + 2),
)
def _sc_gather(f_hbm, idx0_hbm, idx1_hbm, out_hbm, f_sh, idx_v, bufs, *sems):
    gsems = sems[:NBUF]
    hsems = sems[NBUF : 2 * NBUF]
    ssems = sems[2 * NBUF : 3 * NBUF]
    isems = sems[3 * NBUF :]
    cid = lax.axis_index("c")
    sid = lax.axis_index("s")

    def half(idx_hbm, blen, f_base, pos_lo):
        fill = FR // NS
        pltpu.sync_copy(
            f_hbm.at[pl.ds(f_base + sid * fill, fill)],
            f_sh.at[pl.ds(sid * fill, fill)],
        )

        def idx_load(c):
            pltpu.async_copy(
                idx_hbm.at[pl.ds(sid * SEQ_PER_TILE + c * CH, CH)],
                idx_v.at[c % 2],
                isems[c % 2],
            )

        def idx_wait(c):
            pltpu.make_async_copy(
                idx_hbm.at[pl.ds(0, CH)],
                idx_v.at[c % 2],
                isems[c % 2],
            ).wait()

        idx_load(0)
        plsc.subcore_barrier()

        def out_rows(g):
            return pl.ds((sid * SEQ_PER_TILE + g) * S + pos_lo, blen)

        def fetch_start(slot, c, j):
            pltpu.async_copy(
                f_sh.at[idx_v.at[c % 2, j, pl.ds(0, P)]],
                bufs.at[slot, pl.ds(0, P)],
                gsems[slot],
            )
            pltpu.async_copy(
                f_hbm.at[idx_v.at[c % 2, j, pl.ds(P, blen - P)]],
                bufs.at[slot, pl.ds(P, blen - P)],
                hsems[slot],
            )

        def fetch_wait(slot):
            pltpu.make_async_copy(
                f_sh.at[idx_v.at[0, 0, pl.ds(0, P)]],
                bufs.at[slot, pl.ds(0, P)],
                gsems[slot],
            ).wait()
            pltpu.make_async_copy(
                f_hbm.at[idx_v.at[0, 0, pl.ds(P, blen - P)]],
                bufs.at[slot, pl.ds(P, blen - P)],
                hsems[slot],
            ).wait()

        def scatter_start(slot, c, j):
            pltpu.async_copy(
                bufs.at[slot, pl.ds(0, blen)],
                out_hbm.at[out_rows(c * CH + j)],
                ssems[slot],
            )

        def scatter_wait(slot):
            pltpu.make_async_copy(
                bufs.at[slot, pl.ds(0, blen)], out_hbm.at[out_rows(0)], ssems[slot]
            ).wait()

        idx_wait(0)
        idx_load(1)
        for b in range(NBUF):
            fetch_start(b, 0, b)

        for c in range(NCH):
            base = (c * CH) % NBUF
            nfull = (CH - NBUF) // NBUF

            @pl.loop(0, nfull)
            def _(o):
                for k in range(NBUF):
                    slot = (base + k) % NBUF
                    j = o * NBUF + k
                    fetch_wait(slot)
                    scatter_start(slot, c, j)
                    scatter_wait(slot)
                    fetch_start(slot, c, j + NBUF)

            for j in range(nfull * NBUF, CH - NBUF):
                slot = (base + j) % NBUF
                fetch_wait(slot)
                scatter_start(slot, c, j)
                scatter_wait(slot)
                fetch_start(slot, c, j + NBUF)

            if c + 1 < NCH:
                idx_wait(c + 1)
            for j in range(CH - NBUF, CH):
                slot = (base + j) % NBUF
                fetch_wait(slot)
                scatter_start(slot, c, j)
                scatter_wait(slot)
                if c + 1 < NCH:
                    fetch_start(slot, c + 1, j - (CH - NBUF))
            if c + 2 < NCH:
                idx_load(c + 2)

    @pl.when(cid == 0)
    def _():
        half(idx0_hbm, BL0, 0, 0)

    @pl.when(cid == 1)
    def _():
        half(idx1_hbm, BL1, BL0 * VOCAB, BL0)


def kernel(tokens_idx, emb_table):
    f, mask, idx0, idx1 = _build(tokens_idx, emb_table)
    f_flat = f.reshape(S * VOCAB, EMB)
    out = _sc_gather(f_flat, idx0, idx1)
    return out.reshape(B, S, EMB), mask
